# Initial kernel scaffold; baseline (speedup 1.0000x reference)
import jax
import jax.numpy as jnp
from jax.experimental import pallas as pl


def kernel(img_embs, cap_embs, cap_lens, long_cap_embs, long_cap_lens, ln_g, ln_b, w1, b1, w2, b2, scale):
    raise NotImplementedError("write your pallas kernel here")


# speedup vs baseline: 15.3009x; 15.3009x over previous
import jax
import jax.numpy as jnp
from jax.experimental import pallas as pl
from jax.experimental.pallas import tpu as pltpu

_BV, _LS, _D = 32, 576, 512
_MW, _MWL = 60, 140
_HID, _KA, _NK = 102, 115, 288
_NT = 16
_LW = 140
_NEG = -1e30
_IMIN = -(2 ** 31)


def _sortable(x):
    b = jax.lax.bitcast_convert_type(x, jnp.int32)
    return jnp.where(b >= 0, b, jnp.bitwise_xor(jnp.bitwise_not(b), _IMIN))


def _body(lens_s, scal_s, spat, cls, capF, capT, lensv3, lensf, w1, b1, w2,
          b2, lng, lnb, out, hs_s, km_s, ew_s):
    t = pl.program_id(1)

    @pl.when(t == 0)
    def _prep():
        sp = spat[0]
        rn = jnp.sqrt(jnp.sum(sp * sp, axis=1, keepdims=True))
        spn = sp / jnp.maximum(rn, 1e-12)
        glo = jnp.mean(sp, axis=0, keepdims=True)
        glo = glo / jnp.maximum(
            jnp.sqrt(jnp.sum(glo * glo, axis=1, keepdims=True)), 1e-12)

        cf = capF[...]
        wi3 = jax.lax.broadcasted_iota(jnp.int32, (_NT, _LW, _D), 1)
        cfm = jnp.where(wi3 < lensv3[...], cf, 0.0)
        csum = jnp.sum(cfm, axis=1)
        cmean = csum / lensf[...]
        cg = cmean / jnp.maximum(
            jnp.sqrt(jnp.sum(cmean * cmean, axis=1, keepdims=True)), 1e-12)

        q = cg + glo
        scores = jax.lax.dot_general(
            q, spn, (((1,), (1,)), ((), ())),
            preferred_element_type=jnp.float32)

        keys = _sortable(scores)

        def _bis(it, r):
            sh = jnp.left_shift(jnp.int32(1), 31 - it)
            cand = jnp.bitwise_or(r, sh)
            cnt = jnp.sum(
                (keys >= jnp.bitwise_xor(cand, _IMIN)).astype(jnp.int32),
                axis=1, keepdims=True)
            return jnp.where(cnt >= _NK, cand, r)

        r = jax.lax.fori_loop(0, 32, _bis, jnp.zeros((_NT, 1), jnp.int32))
        thr = jnp.bitwise_xor(r, _IMIN)
        kept = keys >= thr
        keptf = kept.astype(jnp.float32)
        km_s[...] = keptf.T

        snk = jnp.where(kept, _NEG, scores)
        mnk = jnp.max(snk, axis=1, keepdims=True)
        e = jnp.exp(snk - mnk) * (1.0 - keptf)
        ew_s[...] = (e / jnp.sum(e, axis=1, keepdims=True)).T

        mu = jnp.mean(sp, axis=1, keepdims=True)
        xm = sp - mu
        var = jnp.mean(xm * xm, axis=1, keepdims=True)
        hn = xm * jax.lax.rsqrt(var + 1e-5) * lng[...] + lnb[...]
        a1 = jnp.dot(hn, w1[...], preferred_element_type=jnp.float32) + b1[...]
        g = 0.5 * a1 * (1.0 + jax.lax.erf(a1 * 0.7071067811865476))
        hs_s[...] = (jnp.dot(g, w2[...], preferred_element_type=jnp.float32)
                     + b2[...]) * scal_s[0]
        out[...] = jnp.zeros_like(out)

    @pl.when(t > 0)
    def _cap():
        i = t - 1
        oh = (jax.lax.broadcasted_iota(jnp.int32, (1, _NT), 1)
              == i).astype(jnp.float32)
        m = jnp.sum(km_s[...] * oh, axis=1, keepdims=True)
        e = jnp.sum(ew_s[...] * oh, axis=1, keepdims=True)
        hs = hs_s[...]
        mx = jnp.max(jnp.where(m > 0, hs, _NEG), axis=0, keepdims=True)
        F = jnp.exp(hs - mx) * m
        A = F / jnp.sum(F, axis=0, keepdims=True)
        E = jnp.concatenate([A, e], axis=1)
        sp = spat[0]
        Y = jax.lax.dot_general(
            E, sp, (((0,), (0,)), ((), ())),
            preferred_element_type=jnp.float32)
        rn = jnp.sqrt(jnp.sum(Y * Y, axis=1, keepdims=True))
        Yn = Y / jnp.maximum(rn, 1e-12)
        c = cls[0]
        cn = c / jnp.maximum(
            jnp.sqrt(jnp.sum(c * c, axis=1, keepdims=True)), 1e-12)
        cap = capT[0]
        capn = cap / jnp.maximum(
            jnp.sqrt(jnp.sum(cap * cap, axis=1, keepdims=True)), 1e-12)
        P = jax.lax.dot_general(
            capn, Yn, (((1,), (1,)), ((), ())),
            preferred_element_type=jnp.float32)
        Pc = jax.lax.dot_general(
            capn, cn, (((1,), (1,)), ((), ())),
            preferred_element_type=jnp.float32)
        mw = jnp.maximum(jnp.max(P, axis=1, keepdims=True), Pc)
        n = lens_s[i]
        wm = (jax.lax.broadcasted_iota(jnp.int32, (_LW, 1), 0)
              < n).astype(jnp.float32)
        simv = jnp.sum(mw * wm) / n.astype(jnp.float32)
        oh8 = (jax.lax.broadcasted_iota(jnp.int32, (1, 8), 1)
               == (i % 8)).astype(jnp.float32)
        out[...] = out[...] + (oh8 * simv)[None]


def kernel(img_embs, cap_embs, cap_lens, long_cap_embs, long_cap_lens,
           ln_g, ln_b, w1, b1, w2, b2, scale):
    spatial = img_embs[:, 1:, :]
    cls = img_embs[:, 0:1, :]
    capF = jnp.concatenate(
        [jnp.pad(cap_embs, ((0, 0), (0, _MWL - _MW), (0, 0))),
         long_cap_embs], axis=0)
    lens = jnp.concatenate([cap_lens, long_cap_lens]).astype(jnp.int32)
    lensv3 = lens[:, None, None]
    lensf = lens.astype(jnp.float32)[:, None]

    out = pl.pallas_call(
        _body,
        grid=(_BV, _NT + 1),
        in_specs=[
            pl.BlockSpec(memory_space=pltpu.SMEM),
            pl.BlockSpec(memory_space=pltpu.SMEM),
            pl.BlockSpec((1, _LS, _D), lambda b, t: (b, 0, 0)),
            pl.BlockSpec((1, 1, _D), lambda b, t: (b, 0, 0)),
            pl.BlockSpec((_NT, _LW, _D), lambda b, t: (0, 0, 0)),
            pl.BlockSpec((1, _LW, _D),
                         lambda b, t: (jnp.maximum(t - 1, 0), 0, 0)),
            pl.BlockSpec((_NT, 1, 1), lambda b, t: (0, 0, 0)),
            pl.BlockSpec((_NT, 1), lambda b, t: (0, 0)),
            pl.BlockSpec((_D, _HID), lambda b, t: (0, 0)),
            pl.BlockSpec((1, _HID), lambda b, t: (0, 0)),
            pl.BlockSpec((_HID, _KA), lambda b, t: (0, 0)),
            pl.BlockSpec((1, _KA), lambda b, t: (0, 0)),
            pl.BlockSpec((1, _D), lambda b, t: (0, 0)),
            pl.BlockSpec((1, _D), lambda b, t: (0, 0)),
        ],
        out_specs=pl.BlockSpec((1, 1, 8), lambda b, t: (b, 0, 0)),
        out_shape=jax.ShapeDtypeStruct((_BV, 1, 8), jnp.float32),
        scratch_shapes=[
            pltpu.VMEM((_LS, _KA), jnp.float32),
            pltpu.VMEM((_LS, _NT), jnp.float32),
            pltpu.VMEM((_LS, _NT), jnp.float32),
        ],
    )(lens, scale.reshape(1), spatial, cls, capF, capF, lensv3, lensf,
      w1, b1[None], w2, b2[None], ln_g[None], ln_b[None])
    return out[:, 0, :]


# speedup vs baseline: 17.4637x; 1.1414x over previous
import jax
import jax.numpy as jnp
from jax.experimental import pallas as pl
from jax.experimental.pallas import tpu as pltpu

_BV, _LS, _D = 32, 576, 512
_MW, _MWL = 60, 140
_HID, _KA, _NK = 102, 115, 288
_NT = 16
_LW = 140
_NEG = -1e30
_IMIN = -(2 ** 31)


def _sortable(x):
    b = jax.lax.bitcast_convert_type(x, jnp.int32)
    return jnp.where(b >= 0, b, jnp.bitwise_xor(jnp.bitwise_not(b), _IMIN))


def _nrm(x, axis=-1):
    return x / jnp.maximum(
        jnp.sqrt(jnp.sum(x * x, axis=axis, keepdims=True)), 1e-12)


def _body(lens_s, scal_s, spat, cls, capF, lensv3, lensf, w1, b1, w2,
          b2, lng, lnb, out, hsT_s, spb_s, km_s, ew_s, cg_s, cT_s):
    b = pl.program_id(0)
    t = pl.program_id(1)

    @pl.when(jnp.logical_and(b == 0, t == 0))
    def _caps():
        cf = capF[...]
        cn3 = _nrm(cf)
        for k in range(_NT):
            cT_s[k] = cn3[k].T
        wi3 = jax.lax.broadcasted_iota(jnp.int32, (_NT, _LW, _D), 1)
        cfm = jnp.where(wi3 < lensv3[...], cf, 0.0)
        cmean = jnp.sum(cfm, axis=1) / lensf[...]
        cg_s[...] = _nrm(cmean)

    @pl.when(t == 0)
    def _prep():
        sp = spat[0]
        spb_s[...] = sp.astype(jnp.bfloat16)
        rn = jnp.sqrt(jnp.sum(sp * sp, axis=1, keepdims=True))
        spn = sp / jnp.maximum(rn, 1e-12)
        glo = _nrm(jnp.mean(sp, axis=0, keepdims=True))

        q = cg_s[...] + glo
        scoresT = jnp.dot(spn, q.T, preferred_element_type=jnp.float32)
        scores = scoresT.T

        keys = _sortable(scores)

        def _bis(it, r):
            sh = jnp.left_shift(jnp.int32(1), 31 - it)
            cand = jnp.bitwise_or(r, sh)
            cnt = jnp.sum(
                (keys >= jnp.bitwise_xor(cand, _IMIN)).astype(jnp.int32),
                axis=1, keepdims=True)
            return jnp.where(cnt >= _NK, cand, r)

        r = jax.lax.fori_loop(0, 32, _bis, jnp.zeros((_NT, 1), jnp.int32))
        thr = jnp.bitwise_xor(r, _IMIN)
        kept = keys >= thr
        keptf = kept.astype(jnp.float32)
        km_s[...] = keptf

        snk = jnp.where(kept, _NEG, scores)
        mnk = jnp.max(snk, axis=1, keepdims=True)
        e = jnp.exp(snk - mnk) * (1.0 - keptf)
        ew_s[...] = e / jnp.sum(e, axis=1, keepdims=True)

        mu = jnp.mean(sp, axis=1, keepdims=True)
        xm = sp - mu
        var = jnp.mean(xm * xm, axis=1, keepdims=True)
        hn = xm * jax.lax.rsqrt(var + 1e-5) * lng[...] + lnb[...]
        a1 = jnp.dot(hn, w1[...], preferred_element_type=jnp.float32) + b1[...]
        g = 0.5 * a1 * (1.0 + jax.lax.erf(a1 * 0.7071067811865476))
        hs = (jnp.dot(g, w2[...], preferred_element_type=jnp.float32)
              + b2[...]) * scal_s[0]
        hsT_s[...] = hs.T
        out[...] = jnp.zeros_like(out)

    @pl.when(t > 0)
    def _cap():
        i = t - 1
        ohc = (jax.lax.broadcasted_iota(jnp.int32, (_NT, 1), 0)
               == i).astype(jnp.float32)
        m_row = jnp.sum(km_s[...] * ohc, axis=0, keepdims=True)
        e_row = jnp.sum(ew_s[...] * ohc, axis=0, keepdims=True)
        hsT = hsT_s[...]
        mx = jnp.max(jnp.where(m_row > 0, hsT, _NEG), axis=1, keepdims=True)
        F = jnp.exp(hsT - mx) * m_row
        A = F / jnp.sum(F, axis=1, keepdims=True)
        E = jnp.concatenate([A, e_row], axis=0)
        Y = jnp.dot(E.astype(jnp.bfloat16), spb_s[...],
                    preferred_element_type=jnp.float32)
        Yn = _nrm(Y)
        clsn = _nrm(cls[0])
        cT = cT_s[i]
        PT = jnp.dot(Yn, cT, preferred_element_type=jnp.float32)
        Pc = jnp.dot(clsn, cT, preferred_element_type=jnp.float32)
        mw = jnp.maximum(jnp.max(PT, axis=0, keepdims=True), Pc)
        n = lens_s[i]
        wm = (jax.lax.broadcasted_iota(jnp.int32, (1, _LW), 1)
              < n).astype(jnp.float32)
        simv = jnp.sum(mw * wm) / n.astype(jnp.float32)
        oh8 = (jax.lax.broadcasted_iota(jnp.int32, (1, 8), 1)
               == (i % 8)).astype(jnp.float32)
        out[...] = out[...] + (oh8 * simv)[None]


def kernel(img_embs, cap_embs, cap_lens, long_cap_embs, long_cap_lens,
           ln_g, ln_b, w1, b1, w2, b2, scale):
    spatial = img_embs[:, 1:, :]
    cls = img_embs[:, 0:1, :]
    capF = jnp.concatenate(
        [jnp.pad(cap_embs, ((0, 0), (0, _MWL - _MW), (0, 0))),
         long_cap_embs], axis=0)
    lens = jnp.concatenate([cap_lens, long_cap_lens]).astype(jnp.int32)
    lensv3 = lens[:, None, None]
    lensf = lens.astype(jnp.float32)[:, None]

    out = pl.pallas_call(
        _body,
        grid=(_BV, _NT + 1),
        in_specs=[
            pl.BlockSpec(memory_space=pltpu.SMEM),
            pl.BlockSpec(memory_space=pltpu.SMEM),
            pl.BlockSpec((1, _LS, _D), lambda b, t: (b, 0, 0)),
            pl.BlockSpec((1, 1, _D), lambda b, t: (b, 0, 0)),
            pl.BlockSpec((_NT, _LW, _D), lambda b, t: (0, 0, 0)),
            pl.BlockSpec((_NT, 1, 1), lambda b, t: (0, 0, 0)),
            pl.BlockSpec((_NT, 1), lambda b, t: (0, 0)),
            pl.BlockSpec((_D, _HID), lambda b, t: (0, 0)),
            pl.BlockSpec((1, _HID), lambda b, t: (0, 0)),
            pl.BlockSpec((_HID, _KA), lambda b, t: (0, 0)),
            pl.BlockSpec((1, _KA), lambda b, t: (0, 0)),
            pl.BlockSpec((1, _D), lambda b, t: (0, 0)),
            pl.BlockSpec((1, _D), lambda b, t: (0, 0)),
        ],
        out_specs=pl.BlockSpec((1, 1, 8), lambda b, t: (b, 0, 0)),
        out_shape=jax.ShapeDtypeStruct((_BV, 1, 8), jnp.float32),
        scratch_shapes=[
            pltpu.VMEM((_KA, _LS), jnp.float32),
            pltpu.VMEM((_LS, _D), jnp.bfloat16),
            pltpu.VMEM((_NT, _LS), jnp.float32),
            pltpu.VMEM((_NT, _LS), jnp.float32),
            pltpu.VMEM((_NT, _D), jnp.float32),
            pltpu.VMEM((_NT, _D, _LW), jnp.float32),
        ],
    )(lens, scale.reshape(1), spatial, cls, capF, lensv3, lensf,
      w1, b1[None], w2, b2[None], ln_g[None], ln_b[None])
    return out[:, 0, :]


# speedup vs baseline: 20.2339x; 1.1586x over previous
import jax
import jax.numpy as jnp
from jax.experimental import pallas as pl
from jax.experimental.pallas import tpu as pltpu

_BV, _LS, _D = 32, 576, 512
_MW, _MWL = 60, 140
_HID, _KA, _NK = 102, 115, 288
_NT = 16
_LW = 140
_NEG = -1e30
_IMIN = -(2 ** 31)


def _sortable(x):
    b = jax.lax.bitcast_convert_type(x, jnp.int32)
    return jnp.where(b >= 0, b, jnp.bitwise_xor(jnp.bitwise_not(b), _IMIN))


def _nrm(x, axis=-1):
    return x / jnp.maximum(
        jnp.sqrt(jnp.sum(x * x, axis=axis, keepdims=True)), 1e-12)


def _body(lens_s, scal_s, spat, cls, capF, lensv3, lensf, w1, b1, w2,
          b2, lng, lnb, out, hsT_s, spb_s, km_s, ew_s, cg_s, cT_s):
    b = pl.program_id(0)
    t = pl.program_id(1)

    @pl.when(jnp.logical_and(b == 0, t == 0))
    def _caps():
        cf = capF[...]
        cn3 = _nrm(cf)
        for k in range(_NT):
            cT_s[k] = cn3[k].T
        wi3 = jax.lax.broadcasted_iota(jnp.int32, (_NT, _LW, _D), 1)
        cfm = jnp.where(wi3 < lensv3[...], cf, 0.0)
        cmean = jnp.sum(cfm, axis=1) / lensf[...]
        cg_s[...] = _nrm(cmean)

    @pl.when(t == 0)
    def _prep():
        sp = spat[0]
        spb_s[...] = sp.astype(jnp.bfloat16)
        rn = jnp.sqrt(jnp.sum(sp * sp, axis=1, keepdims=True))
        spn = sp / jnp.maximum(rn, 1e-12)
        glo = _nrm(jnp.mean(sp, axis=0, keepdims=True))

        q = cg_s[...] + glo
        scoresT = jnp.dot(spn, q.T, preferred_element_type=jnp.float32)
        scores = scoresT.T

        keys = _sortable(scores)

        def _bis(it, r):
            sh = jnp.left_shift(jnp.int32(1), 31 - it)
            cand = jnp.bitwise_or(r, sh)
            cnt = jnp.sum(
                (keys >= jnp.bitwise_xor(cand, _IMIN)).astype(jnp.int32),
                axis=1, keepdims=True)
            return jnp.where(cnt >= _NK, cand, r)

        r = jax.lax.fori_loop(0, 32, _bis, jnp.zeros((_NT, 1), jnp.int32))
        thr = jnp.bitwise_xor(r, _IMIN)
        kept = keys >= thr
        keptf = kept.astype(jnp.float32)
        km_s[...] = keptf

        ew_s[...] = jnp.exp(scores) * (1.0 - keptf)

        mu = jnp.mean(sp, axis=1, keepdims=True)
        xm = sp - mu
        var = jnp.mean(xm * xm, axis=1, keepdims=True)
        hn = xm * jax.lax.rsqrt(var + 1e-5) * lng[...] + lnb[...]
        a1 = jnp.dot(hn, w1[...], preferred_element_type=jnp.float32) + b1[...]
        g = 0.5 * a1 * (1.0 + jax.lax.erf(a1 * 0.7071067811865476))
        hs = (jnp.dot(g, w2[...], preferred_element_type=jnp.float32)
              + b2[...]) * scal_s[0]
        hsT_s[...] = jnp.exp(hs.T)
        out[...] = jnp.zeros_like(out)

    @pl.when(t > 0)
    def _cap():
        i = t - 1
        ohc = (jax.lax.broadcasted_iota(jnp.int32, (_NT, 1), 0)
               == i).astype(jnp.float32)
        m_row = jnp.sum(km_s[...] * ohc, axis=0, keepdims=True)
        e_row = jnp.sum(ew_s[...] * ohc, axis=0, keepdims=True)
        F = hsT_s[...] * m_row
        E = jnp.concatenate([F, e_row], axis=0)
        Y = jnp.dot(E.astype(jnp.bfloat16), spb_s[...],
                    preferred_element_type=jnp.float32)
        Yn = _nrm(Y)
        clsn = _nrm(cls[0])
        cT = cT_s[i]
        PT = jnp.dot(Yn, cT, preferred_element_type=jnp.float32)
        Pc = jnp.dot(clsn, cT, preferred_element_type=jnp.float32)
        mw = jnp.maximum(jnp.max(PT, axis=0, keepdims=True), Pc)
        n = lens_s[i]
        wm = (jax.lax.broadcasted_iota(jnp.int32, (1, _LW), 1)
              < n).astype(jnp.float32)
        simv = jnp.sum(mw * wm) / n.astype(jnp.float32)
        oh8 = (jax.lax.broadcasted_iota(jnp.int32, (1, 8), 1)
               == (i % 8)).astype(jnp.float32)
        out[...] = out[...] + (oh8 * simv)[None]


def kernel(img_embs, cap_embs, cap_lens, long_cap_embs, long_cap_lens,
           ln_g, ln_b, w1, b1, w2, b2, scale):
    spatial = img_embs[:, 1:, :]
    cls = img_embs[:, 0:1, :]
    capF = jnp.concatenate(
        [jnp.pad(cap_embs, ((0, 0), (0, _MWL - _MW), (0, 0))),
         long_cap_embs], axis=0)
    lens = jnp.concatenate([cap_lens, long_cap_lens]).astype(jnp.int32)
    lensv3 = lens[:, None, None]
    lensf = lens.astype(jnp.float32)[:, None]

    out = pl.pallas_call(
        _body,
        grid=(_BV, _NT + 1),
        in_specs=[
            pl.BlockSpec(memory_space=pltpu.SMEM),
            pl.BlockSpec(memory_space=pltpu.SMEM),
            pl.BlockSpec((1, _LS, _D), lambda b, t: (b, 0, 0)),
            pl.BlockSpec((1, 1, _D), lambda b, t: (b, 0, 0)),
            pl.BlockSpec((_NT, _LW, _D), lambda b, t: (0, 0, 0)),
            pl.BlockSpec((_NT, 1, 1), lambda b, t: (0, 0, 0)),
            pl.BlockSpec((_NT, 1), lambda b, t: (0, 0)),
            pl.BlockSpec((_D, _HID), lambda b, t: (0, 0)),
            pl.BlockSpec((1, _HID), lambda b, t: (0, 0)),
            pl.BlockSpec((_HID, _KA), lambda b, t: (0, 0)),
            pl.BlockSpec((1, _KA), lambda b, t: (0, 0)),
            pl.BlockSpec((1, _D), lambda b, t: (0, 0)),
            pl.BlockSpec((1, _D), lambda b, t: (0, 0)),
        ],
        out_specs=pl.BlockSpec((1, 1, 8), lambda b, t: (b, 0, 0)),
        out_shape=jax.ShapeDtypeStruct((_BV, 1, 8), jnp.float32),
        scratch_shapes=[
            pltpu.VMEM((_KA, _LS), jnp.float32),
            pltpu.VMEM((_LS, _D), jnp.bfloat16),
            pltpu.VMEM((_NT, _LS), jnp.float32),
            pltpu.VMEM((_NT, _LS), jnp.float32),
            pltpu.VMEM((_NT, _D), jnp.float32),
            pltpu.VMEM((_NT, _D, _LW), jnp.float32),
        ],
    )(lens, scale.reshape(1), spatial, cls, capF, lensv3, lensf,
      w1, b1[None], w2, b2[None], ln_g[None], ln_b[None])
    return out[:, 0, :]


# speedup vs baseline: 20.4033x; 1.0084x over previous
import functools

import jax
import jax.numpy as jnp
from jax.experimental import pallas as pl
from jax.experimental.pallas import tpu as pltpu
from jax.experimental.pallas import tpu_sc as plsc

_BV, _LS, _D = 32, 576, 512
_MW, _MWL = 60, 140
_HID, _KA, _NK = 102, 115, 288
_NT = 16
_LW = 140
_NR = _BV * _NT
_NW = 32
_RPW = _NR // _NW
_NVR = _LS // 16
_IMIN = -(2 ** 31)


def _sortable(x):
    b = jax.lax.bitcast_convert_type(x, jnp.int32)
    return jnp.where(b >= 0, b, jnp.bitwise_xor(jnp.bitwise_not(b), _IMIN))


def _nrm(x, axis=-1):
    return x / jnp.maximum(
        jnp.sqrt(jnp.sum(x * x, axis=axis, keepdims=True)), 1e-12)


def _score_body(lensv3, lensf, spat, capF, scores, keysT, cg_s):
    b = pl.program_id(0)

    @pl.when(b == 0)
    def _caps():
        cf = capF[...]
        wi3 = jax.lax.broadcasted_iota(jnp.int32, (_NT, _LW, _D), 1)
        cfm = jnp.where(wi3 < lensv3[...], cf, 0.0)
        cmean = jnp.sum(cfm, axis=1) / lensf[...]
        cg_s[...] = _nrm(cmean)

    sp = spat[0]
    rn = jnp.sqrt(jnp.sum(sp * sp, axis=1, keepdims=True))
    spn = sp / jnp.maximum(rn, 1e-12)
    glo = _nrm(jnp.mean(sp, axis=0, keepdims=True))
    q = cg_s[...] + glo
    st = jnp.dot(spn, q.T, preferred_element_type=jnp.float32)
    scores[0] = st.T
    keysT[0] = _sortable(st)


def _tc_scores(spatial, capF, lensv3, lensf):
    return pl.pallas_call(
        _score_body,
        grid=(_BV,),
        in_specs=[
            pl.BlockSpec((_NT, 1, 1), lambda b: (0, 0, 0)),
            pl.BlockSpec((_NT, 1), lambda b: (0, 0)),
            pl.BlockSpec((1, _LS, _D), lambda b: (b, 0, 0)),
            pl.BlockSpec((_NT, _LW, _D), lambda b: (0, 0, 0)),
        ],
        out_specs=[
            pl.BlockSpec((1, _NT, _LS), lambda b: (b, 0, 0)),
            pl.BlockSpec((1, _LS, _NT), lambda b: (b, 0, 0)),
        ],
        out_shape=[
            jax.ShapeDtypeStruct((_BV, _NT, _LS), jnp.float32),
            jax.ShapeDtypeStruct((_BV, _LS, _NT), jnp.int32),
        ],
        scratch_shapes=[pltpu.VMEM((_NT, _D), jnp.float32)],
    )(lensv3, lensf, spatial, capF)


def _sc_sel(keysT):
    mesh = plsc.VectorSubcoreMesh(core_axis_name="c", subcore_axis_name="s")

    @functools.partial(
        pl.kernel,
        out_type=jax.ShapeDtypeStruct((_BV, _NT), jnp.int32),
        mesh=mesh,
        scratch_types=[
            pltpu.VMEM((1, _LS, _NT), jnp.int32),
            pltpu.VMEM((1, _NT), jnp.int32),
        ],
    )
    def run(keys_hbm, thr_hbm, keys_v, thr_v):
        wid = jax.lax.axis_index("s") * 2 + jax.lax.axis_index("c")
        pltpu.sync_copy(keys_hbm.at[pl.ds(wid, 1)], keys_v)

        one = jnp.full((_NT,), 1, jnp.int32)
        zero = jnp.zeros((_NT,), jnp.int32)

        def bis_body(it, carry):
            r, sh = carry
            cand = jnp.bitwise_or(r, sh)
            cands = jnp.bitwise_xor(cand, _IMIN)
            cnt = zero
            for e in range(_LS):
                cnt = cnt + jnp.where(keys_v[0, e] >= cands, one, zero)
            r = jnp.where(cnt >= _NK, cand, r)
            return r, jnp.bitwise_and(jnp.right_shift(sh, 1), 0x7FFFFFFF)

        r, _ = jax.lax.fori_loop(
            0, 32, bis_body,
            (zero, jnp.full((_NT,), _IMIN, jnp.int32)))
        thr_v[0] = jnp.bitwise_xor(r, _IMIN)
        pltpu.sync_copy(thr_v, thr_hbm.at[pl.ds(wid, 1)])

    return run(keysT)


def _main_body(lens_s, scal_s, spat, cls, capF, scores, thr, w1, b1, w2,
               b2, lng, lnb, out, hsT_s, spb_s, km_s, ew_s, cT_s):
    b = pl.program_id(0)
    t = pl.program_id(1)

    @pl.when(jnp.logical_and(b == 0, t == 0))
    def _caps():
        cn3 = _nrm(capF[...])
        for k in range(_NT):
            cT_s[k] = cn3[k].T.astype(jnp.bfloat16)

    @pl.when(t == 0)
    def _prep():
        sp = spat[0]
        spb_s[...] = sp.astype(jnp.bfloat16)
        sc = scores[0]
        keys = _sortable(sc)
        keptf = (keys >= thr[0]).astype(jnp.float32)
        km_s[...] = keptf
        ew_s[...] = jnp.exp(sc) * (1.0 - keptf)

        mu = jnp.mean(sp, axis=1, keepdims=True)
        xm = sp - mu
        var = jnp.mean(xm * xm, axis=1, keepdims=True)
        hn = xm * jax.lax.rsqrt(var + 1e-5) * lng[...] + lnb[...]
        a1 = jnp.dot(hn.astype(jnp.bfloat16), w1[...],
                     preferred_element_type=jnp.float32) + b1[...]
        g = 0.5 * a1 * (1.0 + jax.lax.erf(a1 * 0.7071067811865476))
        hs = (jnp.dot(g.astype(jnp.bfloat16), w2[...],
                      preferred_element_type=jnp.float32)
              + b2[...]) * scal_s[0]
        hsT_s[...] = jnp.exp(hs.T)
        out[...] = jnp.zeros_like(out)

    @pl.when(t > 0)
    def _cap():
        i = t - 1
        ohc = (jax.lax.broadcasted_iota(jnp.int32, (_NT, 1), 0)
               == i).astype(jnp.float32)
        m_row = jnp.sum(km_s[...] * ohc, axis=0, keepdims=True)
        e_row = jnp.sum(ew_s[...] * ohc, axis=0, keepdims=True)
        F = hsT_s[...] * m_row
        E = jnp.concatenate([F, e_row], axis=0)
        Y = jnp.dot(E.astype(jnp.bfloat16), spb_s[...],
                    preferred_element_type=jnp.float32)
        Yn = _nrm(Y)
        clsn = _nrm(cls[0])
        cT = cT_s[i]
        PT = jnp.dot(Yn.astype(jnp.bfloat16), cT,
                     preferred_element_type=jnp.float32)
        Pc = jnp.dot(clsn.astype(jnp.bfloat16), cT,
                     preferred_element_type=jnp.float32)
        mw = jnp.maximum(jnp.max(PT, axis=0, keepdims=True), Pc)
        n = lens_s[i]
        wm = (jax.lax.broadcasted_iota(jnp.int32, (1, _LW), 1)
              < n).astype(jnp.float32)
        simv = jnp.sum(mw * wm) / n.astype(jnp.float32)
        oh8 = (jax.lax.broadcasted_iota(jnp.int32, (1, 8), 1)
               == (i % 8)).astype(jnp.float32)
        out[...] = out[...] + (oh8 * simv)[None]


def kernel(img_embs, cap_embs, cap_lens, long_cap_embs, long_cap_lens,
           ln_g, ln_b, w1, b1, w2, b2, scale):
    spatial = img_embs[:, 1:, :]
    cls = img_embs[:, 0:1, :]
    capF = jnp.concatenate(
        [jnp.pad(cap_embs, ((0, 0), (0, _MWL - _MW), (0, 0))),
         long_cap_embs], axis=0)
    lens = jnp.concatenate([cap_lens, long_cap_lens]).astype(jnp.int32)
    lensv3 = lens[:, None, None]
    lensf = lens.astype(jnp.float32)[:, None]

    scores, keysT = _tc_scores(spatial, capF, lensv3, lensf)
    thr = _sc_sel(keysT).reshape(_BV, _NT, 1)

    out = pl.pallas_call(
        _main_body,
        grid=(_BV, _NT + 1),
        in_specs=[
            pl.BlockSpec(memory_space=pltpu.SMEM),
            pl.BlockSpec(memory_space=pltpu.SMEM),
            pl.BlockSpec((1, _LS, _D), lambda b, t: (b, 0, 0)),
            pl.BlockSpec((1, 1, _D), lambda b, t: (b, 0, 0)),
            pl.BlockSpec((_NT, _LW, _D), lambda b, t: (0, 0, 0)),
            pl.BlockSpec((1, _NT, _LS), lambda b, t: (b, 0, 0)),
            pl.BlockSpec((1, _NT, 1), lambda b, t: (b, 0, 0)),
            pl.BlockSpec((_D, _HID), lambda b, t: (0, 0)),
            pl.BlockSpec((1, _HID), lambda b, t: (0, 0)),
            pl.BlockSpec((_HID, _KA), lambda b, t: (0, 0)),
            pl.BlockSpec((1, _KA), lambda b, t: (0, 0)),
            pl.BlockSpec((1, _D), lambda b, t: (0, 0)),
            pl.BlockSpec((1, _D), lambda b, t: (0, 0)),
        ],
        out_specs=pl.BlockSpec((1, 1, 8), lambda b, t: (b, 0, 0)),
        out_shape=jax.ShapeDtypeStruct((_BV, 1, 8), jnp.float32),
        scratch_shapes=[
            pltpu.VMEM((_KA, _LS), jnp.float32),
            pltpu.VMEM((_LS, _D), jnp.bfloat16),
            pltpu.VMEM((_NT, _LS), jnp.float32),
            pltpu.VMEM((_NT, _LS), jnp.float32),
            pltpu.VMEM((_NT, _D, _LW), jnp.bfloat16),
        ],
    )(lens, scale.reshape(1), spatial, cls, capF, scores, thr,
      w1.astype(jnp.bfloat16), b1[None], w2.astype(jnp.bfloat16), b2[None],
      ln_g[None], ln_b[None])
    return out[:, 0, :]


# speedup vs baseline: 22.6569x; 1.1105x over previous
import functools

import jax
import jax.numpy as jnp
from jax.experimental import pallas as pl
from jax.experimental.pallas import tpu as pltpu
from jax.experimental.pallas import tpu_sc as plsc

_BV, _LS, _D = 32, 576, 512
_MW, _MWL = 60, 140
_HID, _KA, _NK = 102, 115, 288
_NT = 16
_LW = 140
_NR = _BV * _NT
_NW = 32
_RPW = _NR // _NW
_NVR = _LS // 16
_IMIN = -(2 ** 31)


def _sortable(x):
    b = jax.lax.bitcast_convert_type(x, jnp.int32)
    return jnp.where(b >= 0, b, jnp.bitwise_xor(jnp.bitwise_not(b), _IMIN))


def _nrm(x, axis=-1):
    return x / jnp.maximum(
        jnp.sqrt(jnp.sum(x * x, axis=axis, keepdims=True)), 1e-12)


def _score_body(lensv3, lensf, spat, capF, scores, keysT, cg_s):
    b = pl.program_id(0)

    @pl.when(b == 0)
    def _caps():
        cf = capF[...]
        wi3 = jax.lax.broadcasted_iota(jnp.int32, (_NT, _LW, _D), 1)
        cfm = jnp.where(wi3 < lensv3[...], cf, 0.0)
        cmean = jnp.sum(cfm, axis=1) / lensf[...]
        cg_s[...] = _nrm(cmean)

    sp = spat[0]
    rn = jnp.sqrt(jnp.sum(sp * sp, axis=1, keepdims=True))
    spn = sp / jnp.maximum(rn, 1e-12)
    glo = _nrm(jnp.mean(sp, axis=0, keepdims=True))
    q = cg_s[...] + glo
    st = jnp.dot(spn, q.T, preferred_element_type=jnp.float32)
    scores[0] = st.T
    keysT[0] = _sortable(st)


def _tc_scores(spatial, capF, lensv3, lensf):
    return pl.pallas_call(
        _score_body,
        grid=(_BV,),
        in_specs=[
            pl.BlockSpec((_NT, 1, 1), lambda b: (0, 0, 0)),
            pl.BlockSpec((_NT, 1), lambda b: (0, 0)),
            pl.BlockSpec((1, _LS, _D), lambda b: (b, 0, 0)),
            pl.BlockSpec((_NT, _LW, _D), lambda b: (0, 0, 0)),
        ],
        out_specs=[
            pl.BlockSpec((1, _NT, _LS), lambda b: (b, 0, 0)),
            pl.BlockSpec((1, _LS, _NT), lambda b: (b, 0, 0)),
        ],
        out_shape=[
            jax.ShapeDtypeStruct((_BV, _NT, _LS), jnp.float32),
            jax.ShapeDtypeStruct((_BV, _LS, _NT), jnp.int32),
        ],
        scratch_shapes=[pltpu.VMEM((_NT, _D), jnp.float32)],
    )(lensv3, lensf, spatial, capF)


def _sc_sel(keysT):
    mesh = plsc.VectorSubcoreMesh(core_axis_name="c", subcore_axis_name="s")

    @functools.partial(
        pl.kernel,
        out_type=jax.ShapeDtypeStruct((_BV, _NT), jnp.int32),
        mesh=mesh,
        scratch_types=[
            pltpu.VMEM((1, _LS, _NT), jnp.int32),
            pltpu.VMEM((1, _NT), jnp.int32),
        ],
    )
    def run(keys_hbm, thr_hbm, keys_v, thr_v):
        wid = jax.lax.axis_index("s") * 2 + jax.lax.axis_index("c")
        pltpu.sync_copy(keys_hbm.at[pl.ds(wid, 1)], keys_v)

        one = jnp.full((_NT,), 1, jnp.int32)
        zero = jnp.zeros((_NT,), jnp.int32)

        def bis_body(it, carry):
            r, sh = carry
            cand = jnp.bitwise_or(r, sh)
            cands = jnp.bitwise_xor(cand, _IMIN)
            cnt = zero
            for e in range(_LS):
                cnt = cnt + jnp.where(keys_v[0, e] >= cands, one, zero)
            r = jnp.where(cnt >= _NK, cand, r)
            return r, jnp.bitwise_and(jnp.right_shift(sh, 1), 0x7FFFFFFF)

        r, _ = jax.lax.fori_loop(
            0, 32, bis_body,
            (zero, jnp.full((_NT,), _IMIN, jnp.int32)))
        thr_v[0] = jnp.bitwise_xor(r, _IMIN)
        pltpu.sync_copy(thr_v, thr_hbm.at[pl.ds(wid, 1)])

    return run(keysT)


def _main_body(lens_s, scal_s, spat, cls, capF, scores, thr, w1, b1, w2,
               b2, lng, lnb, out, hsT_s, spb_s, km_s, ew_s, cT_s, clsb_s):
    b = pl.program_id(0)
    t = pl.program_id(1)

    @pl.when(jnp.logical_and(b == 0, t == 0))
    def _caps():
        cn3 = _nrm(capF[...])
        for k in range(_NT):
            cT_s[k] = cn3[k].T.astype(jnp.bfloat16)

    @pl.when(t == 0)
    def _prep():
        sp = spat[0]
        spb_s[...] = sp.astype(jnp.bfloat16)
        sc = scores[0]
        keys = _sortable(sc)
        keptf = (keys >= thr[0]).astype(jnp.float32)
        km_s[...] = keptf
        ew_s[...] = jnp.exp(sc) * (1.0 - keptf)

        mu = jnp.mean(sp, axis=1, keepdims=True)
        xm = sp - mu
        var = jnp.mean(xm * xm, axis=1, keepdims=True)
        hn = xm * jax.lax.rsqrt(var + 1e-5) * lng[...] + lnb[...]
        a1 = jnp.dot(hn.astype(jnp.bfloat16), w1[...],
                     preferred_element_type=jnp.float32) + b1[...]
        g = 0.5 * a1 * (1.0 + jax.lax.erf(a1 * 0.7071067811865476))
        hs = (jnp.dot(g.astype(jnp.bfloat16), w2[...],
                      preferred_element_type=jnp.float32)
              + b2[...]) * scal_s[0]
        hsT_s[...] = jnp.exp(hs.T)
        clsb_s[...] = _nrm(cls[0]).astype(jnp.bfloat16)
        out[...] = jnp.zeros_like(out)

    @pl.when(t > 0)
    def _cap():
        i = t - 1
        ohc = (jax.lax.broadcasted_iota(jnp.int32, (_NT, 1), 0)
               == i).astype(jnp.float32)
        m_row = jnp.sum(km_s[...] * ohc, axis=0, keepdims=True)
        e_row = jnp.sum(ew_s[...] * ohc, axis=0, keepdims=True)
        F = hsT_s[...] * m_row
        E = jnp.concatenate([F, e_row], axis=0)
        Y = jnp.dot(E.astype(jnp.bfloat16), spb_s[...],
                    preferred_element_type=jnp.float32)
        rinv = jax.lax.rsqrt(
            jnp.maximum(jnp.sum(Y * Y, axis=1, keepdims=True), 1e-24))
        cT = cT_s[i]
        PT = jnp.dot(Y.astype(jnp.bfloat16), cT,
                     preferred_element_type=jnp.float32) * rinv
        Pc = jnp.dot(clsb_s[...], cT,
                     preferred_element_type=jnp.float32)
        mw = jnp.maximum(jnp.max(PT, axis=0, keepdims=True), Pc)
        n = lens_s[i]
        wm = (jax.lax.broadcasted_iota(jnp.int32, (1, _LW), 1)
              < n).astype(jnp.float32)
        simv = jnp.sum(mw * wm) / n.astype(jnp.float32)
        oh8 = (jax.lax.broadcasted_iota(jnp.int32, (1, 8), 1)
               == (i % 8)).astype(jnp.float32)
        out[...] = out[...] + (oh8 * simv)[None]


def kernel(img_embs, cap_embs, cap_lens, long_cap_embs, long_cap_lens,
           ln_g, ln_b, w1, b1, w2, b2, scale):
    spatial = img_embs[:, 1:, :]
    cls = img_embs[:, 0:1, :]
    capF = jnp.concatenate(
        [jnp.pad(cap_embs, ((0, 0), (0, _MWL - _MW), (0, 0))),
         long_cap_embs], axis=0)
    lens = jnp.concatenate([cap_lens, long_cap_lens]).astype(jnp.int32)
    lensv3 = lens[:, None, None]
    lensf = lens.astype(jnp.float32)[:, None]

    scores, keysT = _tc_scores(spatial, capF, lensv3, lensf)
    thr = _sc_sel(keysT).reshape(_BV, _NT, 1)

    out = pl.pallas_call(
        _main_body,
        grid=(_BV, _NT + 1),
        in_specs=[
            pl.BlockSpec(memory_space=pltpu.SMEM),
            pl.BlockSpec(memory_space=pltpu.SMEM),
            pl.BlockSpec((1, _LS, _D), lambda b, t: (b, 0, 0)),
            pl.BlockSpec((1, 1, _D), lambda b, t: (b, 0, 0)),
            pl.BlockSpec((_NT, _LW, _D), lambda b, t: (0, 0, 0)),
            pl.BlockSpec((1, _NT, _LS), lambda b, t: (b, 0, 0)),
            pl.BlockSpec((1, _NT, 1), lambda b, t: (b, 0, 0)),
            pl.BlockSpec((_D, _HID), lambda b, t: (0, 0)),
            pl.BlockSpec((1, _HID), lambda b, t: (0, 0)),
            pl.BlockSpec((_HID, _KA), lambda b, t: (0, 0)),
            pl.BlockSpec((1, _KA), lambda b, t: (0, 0)),
            pl.BlockSpec((1, _D), lambda b, t: (0, 0)),
            pl.BlockSpec((1, _D), lambda b, t: (0, 0)),
        ],
        out_specs=pl.BlockSpec((1, 1, 8), lambda b, t: (b, 0, 0)),
        out_shape=jax.ShapeDtypeStruct((_BV, 1, 8), jnp.float32),
        scratch_shapes=[
            pltpu.VMEM((_KA, _LS), jnp.float32),
            pltpu.VMEM((_LS, _D), jnp.bfloat16),
            pltpu.VMEM((_NT, _LS), jnp.float32),
            pltpu.VMEM((_NT, _LS), jnp.float32),
            pltpu.VMEM((_NT, _D, _LW), jnp.bfloat16),
            pltpu.VMEM((1, _D), jnp.bfloat16),
        ],
    )(lens, scale.reshape(1), spatial, cls, capF, scores, thr,
      w1.astype(jnp.bfloat16), b1[None], w2.astype(jnp.bfloat16), b2[None],
      ln_g[None], ln_b[None])
    return out[:, 0, :]


# speedup vs baseline: 22.8157x; 1.0070x over previous
import functools

import jax
import jax.numpy as jnp
from jax.experimental import pallas as pl
from jax.experimental.pallas import tpu as pltpu
from jax.experimental.pallas import tpu_sc as plsc

_BV, _LS, _D = 32, 576, 512
_MW, _MWL = 60, 140
_HID, _KA, _NK = 102, 115, 288
_NT = 16
_LW = 140
_NR = _BV * _NT
_NW = 32
_RPW = _NR // _NW
_NVR = _LS // 16
_IMIN = -(2 ** 31)


def _sortable(x):
    b = jax.lax.bitcast_convert_type(x, jnp.int32)
    return jnp.where(b >= 0, b, jnp.bitwise_xor(jnp.bitwise_not(b), _IMIN))


def _nrm(x, axis=-1):
    return x / jnp.maximum(
        jnp.sqrt(jnp.sum(x * x, axis=axis, keepdims=True)), 1e-12)


def _score_body(lensv3, lensf, spat, capF, scores, keysT, cg_s):
    b = pl.program_id(0)

    @pl.when(b == 0)
    def _caps():
        cf = capF[...]
        wi3 = jax.lax.broadcasted_iota(jnp.int32, (_NT, _LW, _D), 1)
        cfm = jnp.where(wi3 < lensv3[...], cf, 0.0)
        cmean = jnp.sum(cfm, axis=1) / lensf[...]
        cg_s[...] = _nrm(cmean)

    sp = spat[0]
    rn = jnp.sqrt(jnp.sum(sp * sp, axis=1, keepdims=True))
    spn = sp / jnp.maximum(rn, 1e-12)
    glo = _nrm(jnp.mean(sp, axis=0, keepdims=True))
    q = cg_s[...] + glo
    st = jnp.dot(spn, q.T, preferred_element_type=jnp.float32)
    scores[0] = st.T
    keysT[0] = _sortable(st)


def _tc_scores(spatial, capF, lensv3, lensf):
    return pl.pallas_call(
        _score_body,
        grid=(_BV,),
        in_specs=[
            pl.BlockSpec((_NT, 1, 1), lambda b: (0, 0, 0)),
            pl.BlockSpec((_NT, 1), lambda b: (0, 0)),
            pl.BlockSpec((1, _LS, _D), lambda b: (b, 0, 0)),
            pl.BlockSpec((_NT, _LW, _D), lambda b: (0, 0, 0)),
        ],
        out_specs=[
            pl.BlockSpec((1, _NT, _LS), lambda b: (b, 0, 0)),
            pl.BlockSpec((1, _LS, _NT), lambda b: (b, 0, 0)),
        ],
        out_shape=[
            jax.ShapeDtypeStruct((_BV, _NT, _LS), jnp.float32),
            jax.ShapeDtypeStruct((_BV, _LS, _NT), jnp.int32),
        ],
        scratch_shapes=[pltpu.VMEM((_NT, _D), jnp.float32)],
    )(lensv3, lensf, spatial, capF)


def _sc_sel(keysT):
    mesh = plsc.VectorSubcoreMesh(core_axis_name="c", subcore_axis_name="s")

    @functools.partial(
        pl.kernel,
        out_type=jax.ShapeDtypeStruct((_BV, _NT), jnp.int32),
        mesh=mesh,
        scratch_types=[
            pltpu.VMEM((1, _LS, _NT), jnp.int32),
            pltpu.VMEM((1, _NT), jnp.int32),
        ],
    )
    def run(keys_hbm, thr_hbm, keys_v, thr_v):
        wid = jax.lax.axis_index("s") * 2 + jax.lax.axis_index("c")
        pltpu.sync_copy(keys_hbm.at[pl.ds(wid, 1)], keys_v)

        one = jnp.full((_NT,), 1, jnp.int32)
        zero = jnp.zeros((_NT,), jnp.int32)

        def bis_body(it, carry):
            r, sh = carry
            cand = jnp.bitwise_or(r, sh)
            cands = jnp.bitwise_xor(cand, _IMIN)
            cnt = zero
            for e in range(_LS):
                cnt = cnt + jnp.where(keys_v[0, e] >= cands, one, zero)
            r = jnp.where(cnt >= _NK, cand, r)
            return r, jnp.bitwise_and(jnp.right_shift(sh, 1), 0x7FFFFFFF)

        r, _ = jax.lax.fori_loop(
            0, 32, bis_body,
            (zero, jnp.full((_NT,), _IMIN, jnp.int32)))
        thr_v[0] = jnp.bitwise_xor(r, _IMIN)
        pltpu.sync_copy(thr_v, thr_hbm.at[pl.ds(wid, 1)])

    return run(keysT)


def _main_body(lens_s, scal_s, spat, cls, capF, scores, thr, w1, b1, w2,
               b2, lng, lnb, out, hsT_s, spb_s, km_s, ew_s, cT_s, clsb_s):
    b = pl.program_id(0)
    t = pl.program_id(1)

    @pl.when(jnp.logical_and(b == 0, t == 0))
    def _caps():
        cn3 = _nrm(capF[...])
        for k in range(_NT):
            cT_s[k] = cn3[k].T.astype(jnp.bfloat16)

    @pl.when(t == 0)
    def _prep():
        sp = spat[0]
        spb_s[...] = sp.astype(jnp.bfloat16)
        sc = scores[0]
        keys = _sortable(sc)
        keptf = (keys >= thr[0]).astype(jnp.float32)
        km_s[...] = keptf
        ew_s[...] = jnp.exp(sc) * (1.0 - keptf)

        mu = jnp.mean(sp, axis=1, keepdims=True)
        xm = sp - mu
        var = jnp.mean(xm * xm, axis=1, keepdims=True)
        hn = xm * jax.lax.rsqrt(var + 1e-5) * lng[...] + lnb[...]
        a1 = jnp.dot(hn.astype(jnp.bfloat16), w1[...],
                     preferred_element_type=jnp.float32) + b1[...]
        g = 0.5 * a1 * (1.0 + jax.lax.erf(a1 * 0.7071067811865476))
        hs = (jnp.dot(g.astype(jnp.bfloat16), w2[...],
                      preferred_element_type=jnp.float32)
              + b2[...]) * scal_s[0]
        hsT_s[...] = jnp.exp(hs.T)
        clsb_s[...] = _nrm(cls[0]).astype(jnp.bfloat16)
        out[...] = jnp.zeros_like(out)

    @pl.when(t > 0)
    def _cap():
        i = t - 1
        m_row = km_s[pl.ds(i, 1)]
        e_row = ew_s[pl.ds(i, 1)]
        F = hsT_s[...] * m_row
        E = jnp.concatenate([F, e_row], axis=0)
        Y = jnp.dot(E.astype(jnp.bfloat16), spb_s[...],
                    preferred_element_type=jnp.float32)
        rinv = jax.lax.rsqrt(
            jnp.maximum(jnp.sum(Y * Y, axis=1, keepdims=True), 1e-24))
        cT = cT_s[i]
        PT = jnp.dot(Y.astype(jnp.bfloat16), cT,
                     preferred_element_type=jnp.float32) * rinv
        Pc = jnp.dot(clsb_s[...], cT,
                     preferred_element_type=jnp.float32)
        mw = jnp.maximum(jnp.max(PT, axis=0, keepdims=True), Pc)
        n = lens_s[i]
        wm = (jax.lax.broadcasted_iota(jnp.int32, (1, _LW), 1)
              < n).astype(jnp.float32)
        simv = jnp.sum(mw * wm) / n.astype(jnp.float32)
        oh8 = (jax.lax.broadcasted_iota(jnp.int32, (1, 8), 1)
               == (i % 8)).astype(jnp.float32)
        out[...] = out[...] + (oh8 * simv)[None]


def kernel(img_embs, cap_embs, cap_lens, long_cap_embs, long_cap_lens,
           ln_g, ln_b, w1, b1, w2, b2, scale):
    spatial = img_embs[:, 1:, :]
    cls = img_embs[:, 0:1, :]
    capF = jnp.concatenate(
        [jnp.pad(cap_embs, ((0, 0), (0, _MWL - _MW), (0, 0))),
         long_cap_embs], axis=0)
    lens = jnp.concatenate([cap_lens, long_cap_lens]).astype(jnp.int32)
    lensv3 = lens[:, None, None]
    lensf = lens.astype(jnp.float32)[:, None]

    scores, keysT = _tc_scores(spatial, capF, lensv3, lensf)
    thr = _sc_sel(keysT).reshape(_BV, _NT, 1)

    out = pl.pallas_call(
        _main_body,
        grid=(_BV, _NT + 1),
        in_specs=[
            pl.BlockSpec(memory_space=pltpu.SMEM),
            pl.BlockSpec(memory_space=pltpu.SMEM),
            pl.BlockSpec((1, _LS, _D), lambda b, t: (b, 0, 0)),
            pl.BlockSpec((1, 1, _D), lambda b, t: (b, 0, 0)),
            pl.BlockSpec((_NT, _LW, _D), lambda b, t: (0, 0, 0)),
            pl.BlockSpec((1, _NT, _LS), lambda b, t: (b, 0, 0)),
            pl.BlockSpec((1, _NT, 1), lambda b, t: (b, 0, 0)),
            pl.BlockSpec((_D, _HID), lambda b, t: (0, 0)),
            pl.BlockSpec((1, _HID), lambda b, t: (0, 0)),
            pl.BlockSpec((_HID, _KA), lambda b, t: (0, 0)),
            pl.BlockSpec((1, _KA), lambda b, t: (0, 0)),
            pl.BlockSpec((1, _D), lambda b, t: (0, 0)),
            pl.BlockSpec((1, _D), lambda b, t: (0, 0)),
        ],
        out_specs=pl.BlockSpec((1, 1, 8), lambda b, t: (b, 0, 0)),
        out_shape=jax.ShapeDtypeStruct((_BV, 1, 8), jnp.float32),
        scratch_shapes=[
            pltpu.VMEM((_KA, _LS), jnp.float32),
            pltpu.VMEM((_LS, _D), jnp.bfloat16),
            pltpu.VMEM((_NT, _LS), jnp.float32),
            pltpu.VMEM((_NT, _LS), jnp.float32),
            pltpu.VMEM((_NT, _D, _LW), jnp.bfloat16),
            pltpu.VMEM((1, _D), jnp.bfloat16),
        ],
    )(lens, scale.reshape(1), spatial, cls, capF, scores, thr,
      w1.astype(jnp.bfloat16), b1[None], w2.astype(jnp.bfloat16), b2[None],
      ln_g[None], ln_b[None])
    return out[:, 0, :]
